# baseline (speedup 1.0000x reference)
import functools

import jax
import jax.numpy as jnp
from jax import lax
from jax.experimental import pallas as pl
from jax.experimental.pallas import tpu as pltpu
from jax.experimental.pallas import tpu_sc as plsc

B = 16384
D = 64
NUM_WORKERS = 32
ROWS_PER_W = B // NUM_WORKERS
CHUNK = 128
CHUNKS_PER_W = ROWS_PER_W // CHUNK


def _sc_gather(node_emb, rel_emb, hidx2d, ridx2d, tidx2d):
    mesh = plsc.VectorSubcoreMesh(core_axis_name="c", subcore_axis_name="s")
    out_sds = jax.ShapeDtypeStruct((B, D), jnp.float32)

    @functools.partial(
        pl.kernel,
        out_type=[out_sds, out_sds, out_sds],
        mesh=mesh,
        compiler_params=pltpu.CompilerParams(use_tc_tiling_on_sc=False),
        scratch_types=[
            pltpu.VMEM((CHUNKS_PER_W, CHUNK), jnp.int32),
            pltpu.VMEM((CHUNKS_PER_W, CHUNK), jnp.int32),
            pltpu.VMEM((CHUNKS_PER_W, CHUNK), jnp.int32),
            pltpu.VMEM((ROWS_PER_W, D), jnp.float32),
            pltpu.VMEM((ROWS_PER_W, D), jnp.float32),
            pltpu.VMEM((ROWS_PER_W, D), jnp.float32),
            pltpu.SemaphoreType.DMA,
        ],
    )
    def k(node_hbm, rel_hbm, hidx_hbm, ridx_hbm, tidx_hbm,
          head_out, rel_out, tail_out,
          hidx_v, ridx_v, tidx_v, hrows, rrows, trows, sem):
        wid = lax.axis_index("s") * 2 + lax.axis_index("c")
        idx_base = wid * CHUNKS_PER_W
        pltpu.sync_copy(hidx_hbm.at[pl.ds(idx_base, CHUNKS_PER_W)], hidx_v)
        pltpu.sync_copy(ridx_hbm.at[pl.ds(idx_base, CHUNKS_PER_W)], ridx_v)
        pltpu.sync_copy(tidx_hbm.at[pl.ds(idx_base, CHUNKS_PER_W)], tidx_v)
        copies = []
        for j in range(CHUNKS_PER_W):
            sl = pl.ds(j * CHUNK, CHUNK)
            copies.append(pltpu.async_copy(node_hbm.at[hidx_v.at[j]], hrows.at[sl], sem))
            copies.append(pltpu.async_copy(node_hbm.at[tidx_v.at[j]], trows.at[sl], sem))
            copies.append(pltpu.async_copy(rel_hbm.at[ridx_v.at[j]], rrows.at[sl], sem))
        for c in copies:
            c.wait()
        row_base = wid * ROWS_PER_W
        pltpu.sync_copy(hrows, head_out.at[pl.ds(row_base, ROWS_PER_W)])
        pltpu.sync_copy(rrows, rel_out.at[pl.ds(row_base, ROWS_PER_W)])
        pltpu.sync_copy(trows, tail_out.at[pl.ds(row_base, ROWS_PER_W)])

    return k(node_emb, rel_emb, hidx2d, ridx2d, tidx2d)


TC_BLK = 2048


def _tc_body(h_ref, r_ref, t_ref, o_ref):
    h = h_ref[...]
    r = r_ref[...]
    t = t_ref[...]
    hh = jnp.sum(h * h, axis=1, keepdims=True)
    tt = jnp.sum(t * t, axis=1, keepdims=True)
    inv_h = 1.0 / jnp.maximum(jnp.sqrt(hh), 1e-12)
    inv_t = 1.0 / jnp.maximum(jnp.sqrt(tt), 1e-12)
    d = h * inv_h + r - t * inv_t
    o_ref[...] = jnp.sqrt(jnp.sum(d * d, axis=1))


def _tc_compute(heads, rels, tails):
    return pl.pallas_call(
        _tc_body,
        grid=(B // TC_BLK,),
        in_specs=[
            pl.BlockSpec((TC_BLK, D), lambda i: (i, 0)),
            pl.BlockSpec((TC_BLK, D), lambda i: (i, 0)),
            pl.BlockSpec((TC_BLK, D), lambda i: (i, 0)),
        ],
        out_specs=pl.BlockSpec((TC_BLK,), lambda i: (i,)),
        out_shape=jax.ShapeDtypeStruct((B,), jnp.float32),
    )(heads, rels, tails)


def kernel(head_index, rel_type, tail_index, node_emb, rel_emb):
    hidx = head_index.astype(jnp.int32).reshape(NUM_WORKERS * CHUNKS_PER_W, CHUNK)
    ridx = rel_type.astype(jnp.int32).reshape(NUM_WORKERS * CHUNKS_PER_W, CHUNK)
    tidx = tail_index.astype(jnp.int32).reshape(NUM_WORKERS * CHUNKS_PER_W, CHUNK)
    heads, rels, tails = _sc_gather(node_emb, rel_emb, hidx, ridx, tidx)
    return _tc_compute(heads, rels, tails)


# speedup vs baseline: 1.0141x; 1.0141x over previous
import dataclasses
import functools

import jax
import jax.numpy as jnp
from jax import lax
from jax.experimental import pallas as pl
from jax.experimental.pallas import tpu as pltpu
from jax.experimental.pallas import tpu_sc as plsc

B = 16384
D = 64
NW = 32
ROWS_PER_W = B // NW
CHUNK = 128
NCHUNK = ROWS_PER_W // CHUNK
GROUPS_PER_CHUNK = CHUNK // 16
PSTRIDE = 97

_MAGIC = 0x5F3759DF
_BIG = 1e12


def _sc_compiler_params():
    cp = pltpu.CompilerParams()
    if "needs_layout_passes" in pltpu.CompilerParams.__dataclass_fields__:
        cp = dataclasses.replace(cp, needs_layout_passes=False)
    return cp


def _rsqrt16(x):
    i = plsc.bitcast(x, jnp.int32)
    y = plsc.bitcast(_MAGIC - lax.shift_right_logical(i, 1), jnp.float32)
    xh = x * 0.5
    for _ in range(3):
        y = y * (1.5 - xh * y * y)
    return y


def kernel(head_index, rel_type, tail_index, node_emb, rel_emb):
    nodes2 = node_emb.reshape(500000, 128)
    rel2 = rel_emb.reshape(500, 128)

    def split(idx):
        idx = idx.astype(jnp.int32)
        pair = lax.shift_right_logical(idx, 1).reshape(NW * NCHUNK, CHUNK)
        off = lax.shift_left(jnp.bitwise_and(idx, 1), 6).reshape(NW * NCHUNK, CHUNK)
        return pair, off

    hp, ho = split(head_index)
    rp, ro = split(rel_type)
    tp, to = split(tail_index)
    mesh = plsc.VectorSubcoreMesh(core_axis_name="c", subcore_axis_name="s")

    @functools.partial(
        pl.kernel,
        out_type=jax.ShapeDtypeStruct((B,), jnp.float32),
        mesh=mesh,
        compiler_params=_sc_compiler_params(),
        scratch_types=[
            pltpu.VMEM((NCHUNK, CHUNK), jnp.int32),
            pltpu.VMEM((NCHUNK, CHUNK), jnp.int32),
            pltpu.VMEM((NCHUNK, CHUNK), jnp.int32),
            pltpu.VMEM((NCHUNK, CHUNK), jnp.int32),
            pltpu.VMEM((NCHUNK, CHUNK), jnp.int32),
            pltpu.VMEM((NCHUNK, CHUNK), jnp.int32),
            pltpu.VMEM((2, CHUNK, 128), jnp.float32),
            pltpu.VMEM((2, CHUNK, 128), jnp.float32),
            pltpu.VMEM((2, CHUNK, 128), jnp.float32),
            pltpu.VMEM((16, PSTRIDE), jnp.float32),
            pltpu.VMEM((ROWS_PER_W,), jnp.float32),
            pltpu.SemaphoreType.DMA,
            pltpu.SemaphoreType.DMA,
        ],
    )
    def k(nodes_hbm, rel_hbm, hp_hbm, rp_hbm, tp_hbm, ho_hbm, ro_hbm, to_hbm,
          out_hbm,
          hp_v, rp_v, tp_v, ho_v, ro_v, to_v, hbuf, rbuf, tbuf,
          part, out_v, sem0, sem1):
        wid = lax.axis_index("s") * 2 + lax.axis_index("c")
        idx_base = wid * NCHUNK
        pltpu.sync_copy(hp_hbm.at[pl.ds(idx_base, NCHUNK)], hp_v)
        pltpu.sync_copy(rp_hbm.at[pl.ds(idx_base, NCHUNK)], rp_v)
        pltpu.sync_copy(tp_hbm.at[pl.ds(idx_base, NCHUNK)], tp_v)
        pltpu.sync_copy(ho_hbm.at[pl.ds(idx_base, NCHUNK)], ho_v)
        pltpu.sync_copy(ro_hbm.at[pl.ds(idx_base, NCHUNK)], ro_v)
        pltpu.sync_copy(to_hbm.at[pl.ds(idx_base, NCHUNK)], to_v)

        sems = (sem0, sem1)

        def start_chunk(j, buf):
            sem = sems[buf]
            return (
                pltpu.async_copy(nodes_hbm.at[hp_v.at[j]], hbuf.at[buf], sem),
                pltpu.async_copy(rel_hbm.at[rp_v.at[j]], rbuf.at[buf], sem),
                pltpu.async_copy(nodes_hbm.at[tp_v.at[j]], tbuf.at[buf], sem),
            )

        lanes = lax.iota(jnp.int32, 16)

        def compute_chunk(j, buf):
            @pl.loop(0, GROUPS_PER_CHUNK)
            def _(g):
                base = g * 16
                offh = ho_v[j, pl.ds(base, 16)]
                offr = ro_v[j, pl.ds(base, 16)]
                offt = to_v[j, pl.ds(base, 16)]
                for i in range(16):
                    row = base + i
                    l0h = offh[i]
                    l0r = offr[i]
                    l0t = offt[i]
                    hh = tt = rr = hr = ht = rt = None
                    for kk in range(4):
                        h = hbuf[buf, row, pl.ds(l0h + kk * 16, 16)]
                        r = rbuf[buf, row, pl.ds(l0r + kk * 16, 16)]
                        t = tbuf[buf, row, pl.ds(l0t + kk * 16, 16)]
                        if kk == 0:
                            hh, tt, rr = h * h, t * t, r * r
                            hr, ht, rt = h * r, h * t, r * t
                        else:
                            hh += h * h
                            tt += t * t
                            rr += r * r
                            hr += h * r
                            ht += h * t
                            rt += r * t
                    part[i, pl.ds(0, 16)] = hh
                    part[i, pl.ds(16, 16)] = tt
                    part[i, pl.ds(32, 16)] = rr
                    part[i, pl.ds(48, 16)] = hr
                    part[i, pl.ds(64, 16)] = ht
                    part[i, pl.ds(80, 16)] = rt

                sums = []
                for q in range(6):
                    acc = plsc.load_gather(
                        part, [lanes, jnp.full((16,), q * 16, jnp.int32)])
                    for c in range(1, 16):
                        acc += plsc.load_gather(
                            part, [lanes, jnp.full((16,), q * 16 + c, jnp.int32)])
                    sums.append(acc)
                hh, tt, rr, hr, ht, rt = sums

                a = jnp.minimum(_rsqrt16(hh), _BIG)
                b = jnp.minimum(_rsqrt16(tt), _BIG)
                dd = hh * a * a + rr + tt * b * b + 2.0 * (
                    hr * a - ht * (a * b) - rt * b)
                dd = jnp.maximum(dd, 0.0)
                out_v[pl.ds(j * CHUNK + base, 16)] = dd * jnp.minimum(
                    _rsqrt16(dd), _BIG)

        pending = start_chunk(0, 0)
        for j in range(NCHUNK):
            for c in pending:
                c.wait()
            if j + 1 < NCHUNK:
                nxt = start_chunk(j + 1, (j + 1) % 2)
            compute_chunk(j, j % 2)
            if j + 1 < NCHUNK:
                pending = nxt

        pltpu.sync_copy(out_v, out_hbm.at[pl.ds(wid * ROWS_PER_W, ROWS_PER_W)])

    return k(nodes2, rel2, hp, rp, tp, ho, ro, to)


# speedup vs baseline: 1.4729x; 1.4524x over previous
import dataclasses
import functools

import jax
import jax.numpy as jnp
from jax import lax
from jax.experimental import pallas as pl
from jax.experimental.pallas import tpu as pltpu
from jax.experimental.pallas import tpu_sc as plsc

B = 16384
NW = 32
ROWS_PER_W = B // NW
GROUPS = ROWS_PER_W // 16
PSTRIDE = 97

_MAGIC = 0x5F3759DF
_BIG = 1e12


def _sc_compiler_params():
    cp = pltpu.CompilerParams()
    if "needs_layout_passes" in pltpu.CompilerParams.__dataclass_fields__:
        cp = dataclasses.replace(cp, needs_layout_passes=False)
    return cp


def _rsqrt16(x):
    i = plsc.bitcast(x, jnp.int32)
    y = plsc.bitcast(_MAGIC - lax.shift_right_logical(i, 1), jnp.float32)
    xh = x * 0.5
    for _ in range(3):
        y = y * (1.5 - xh * y * y)
    return y


def kernel(head_index, rel_type, tail_index, node_emb, rel_emb):
    hidx = head_index.astype(jnp.int32).reshape(NW, ROWS_PER_W)
    tidx = tail_index.astype(jnp.int32).reshape(NW, ROWS_PER_W)
    ridx = rel_type.astype(jnp.int32).reshape(NW, ROWS_PER_W)
    mesh = plsc.VectorSubcoreMesh(core_axis_name="c", subcore_axis_name="s")

    @functools.partial(
        pl.kernel,
        out_type=jax.ShapeDtypeStruct((B,), jnp.float32),
        mesh=mesh,
        compiler_params=_sc_compiler_params(),
        scratch_types=[
            pltpu.VMEM((ROWS_PER_W,), jnp.int32),
            pltpu.VMEM((ROWS_PER_W,), jnp.int32),
            pltpu.VMEM((ROWS_PER_W,), jnp.int32),
            pltpu.VMEM((2, 16, 8, 64), jnp.float32),
            pltpu.VMEM((2, 16, 8, 64), jnp.float32),
            pltpu.VMEM((2, 16, 8, 64), jnp.float32),
            pltpu.VMEM((16, PSTRIDE), jnp.float32),
            pltpu.VMEM((ROWS_PER_W,), jnp.float32),
            pltpu.SemaphoreType.DMA,
            pltpu.SemaphoreType.DMA,
        ],
    )
    def k(nodes_hbm, rel_hbm, hidx_hbm, tidx_hbm, ridx_hbm, out_hbm,
          h_v, t_v, r_v, hw, tw, rw, part, out_v, sem0, sem1):
        wid = lax.axis_index("s") * 2 + lax.axis_index("c")
        pltpu.sync_copy(hidx_hbm.at[wid], h_v)
        pltpu.sync_copy(tidx_hbm.at[wid], t_v)
        pltpu.sync_copy(ridx_hbm.at[wid], r_v)

        lanes = lax.iota(jnp.int32, 16)
        sems = (sem0, sem1)

        def issue_group(g, buf):
            sem = sems[buf]
            sl16 = pl.ds(g * 16, 16)
            hs = lax.shift_left(lax.shift_right_logical(h_v[sl16], 3), 3)
            ts = lax.shift_left(lax.shift_right_logical(t_v[sl16], 3), 3)
            rs = lax.shift_left(lax.shift_right_logical(r_v[sl16], 3), 3)
            for k2 in range(16):
                pltpu.async_copy(
                    nodes_hbm.at[pl.ds(pl.multiple_of(hs[k2], 8), 8)],
                    hw.at[buf, k2], sem)
                pltpu.async_copy(
                    nodes_hbm.at[pl.ds(pl.multiple_of(ts[k2], 8), 8)],
                    tw.at[buf, k2], sem)
                pltpu.async_copy(
                    rel_hbm.at[pl.ds(pl.multiple_of(rs[k2], 8), 8)],
                    rw.at[buf, k2], sem)

        def drain_group(buf):
            sem = sems[buf]
            for k2 in range(16):
                pltpu.make_async_copy(
                    nodes_hbm.at[pl.ds(0, 8)], hw.at[buf, k2], sem).wait()
                pltpu.make_async_copy(
                    nodes_hbm.at[pl.ds(0, 8)], tw.at[buf, k2], sem).wait()
                pltpu.make_async_copy(
                    rel_hbm.at[pl.ds(0, 8)], rw.at[buf, k2], sem).wait()

        def compute_group(g, buf):
            sl16 = pl.ds(g * 16, 16)
            hsub = jnp.bitwise_and(h_v[sl16], 7)
            tsub = jnp.bitwise_and(t_v[sl16], 7)
            rsub = jnp.bitwise_and(r_v[sl16], 7)
            for k2 in range(16):
                sh = hsub[k2]
                st = tsub[k2]
                sr = rsub[k2]
                hh = tt = rr = hr = ht = rt = None
                for kk in range(4):
                    sl = pl.ds(kk * 16, 16)
                    h = hw[buf, k2, sh, sl]
                    t = tw[buf, k2, st, sl]
                    r = rw[buf, k2, sr, sl]
                    if kk == 0:
                        hh, tt, rr = h * h, t * t, r * r
                        hr, ht, rt = h * r, h * t, r * t
                    else:
                        hh += h * h
                        tt += t * t
                        rr += r * r
                        hr += h * r
                        ht += h * t
                        rt += r * t
                part[k2, pl.ds(0, 16)] = hh
                part[k2, pl.ds(16, 16)] = tt
                part[k2, pl.ds(32, 16)] = rr
                part[k2, pl.ds(48, 16)] = hr
                part[k2, pl.ds(64, 16)] = ht
                part[k2, pl.ds(80, 16)] = rt

            sums = []
            for q in range(6):
                acc = plsc.load_gather(
                    part, [lanes, jnp.full((16,), q * 16, jnp.int32)])
                for c in range(1, 16):
                    acc += plsc.load_gather(
                        part, [lanes, jnp.full((16,), q * 16 + c, jnp.int32)])
                sums.append(acc)
            shh, stt, srr, shr, sht, srt = sums

            a = jnp.minimum(_rsqrt16(shh), _BIG)
            b = jnp.minimum(_rsqrt16(stt), _BIG)
            dd = shh * a * a + srr + stt * b * b + 2.0 * (
                shr * a - sht * (a * b) - srt * b)
            dd = jnp.maximum(dd, 0.0)
            out_v[pl.ds(g * 16, 16)] = dd * jnp.minimum(_rsqrt16(dd), _BIG)

        issue_group(0, 0)

        @pl.loop(0, GROUPS // 2)
        def _(p):
            g0 = p * 2
            drain_group(0)
            issue_group(g0 + 1, 1)
            compute_group(g0, 0)
            drain_group(1)

            @pl.when(p < GROUPS // 2 - 1)
            def _():
                issue_group(g0 + 2, 0)

            compute_group(g0 + 1, 1)

        pltpu.sync_copy(out_v, out_hbm.at[pl.ds(wid * ROWS_PER_W, ROWS_PER_W)])

    return k(node_emb, rel_emb, hidx, tidx, ridx)


# speedup vs baseline: 1.5354x; 1.0424x over previous
import dataclasses
import functools

import jax
import jax.numpy as jnp
from jax import lax
from jax.experimental import pallas as pl
from jax.experimental.pallas import tpu as pltpu
from jax.experimental.pallas import tpu_sc as plsc

B = 16384
NW = 32
ROWS_PER_W = B // NW
GROUPS = ROWS_PER_W // 16
PSTRIDE = 97

_MAGIC = 0x5F3759DF
_BIG = 1e12


def _sc_compiler_params():
    cp = pltpu.CompilerParams()
    if "needs_layout_passes" in pltpu.CompilerParams.__dataclass_fields__:
        cp = dataclasses.replace(cp, needs_layout_passes=False)
    return cp


def _rsqrt16(x):
    i = plsc.bitcast(x, jnp.int32)
    y = plsc.bitcast(_MAGIC - lax.shift_right_logical(i, 1), jnp.float32)
    xh = x * 0.5
    for _ in range(3):
        y = y * (1.5 - xh * y * y)
    return y


def kernel(head_index, rel_type, tail_index, node_emb, rel_emb):
    rel2 = rel_emb.reshape(500, 128)
    hidx = head_index.astype(jnp.int32).reshape(NW, ROWS_PER_W)
    tidx = tail_index.astype(jnp.int32).reshape(NW, ROWS_PER_W)
    r32 = rel_type.astype(jnp.int32)
    rp = lax.shift_right_logical(r32, 1).reshape(NW, ROWS_PER_W)
    ro = lax.shift_left(jnp.bitwise_and(r32, 1), 6).reshape(NW, ROWS_PER_W)
    mesh = plsc.VectorSubcoreMesh(core_axis_name="c", subcore_axis_name="s")

    @functools.partial(
        pl.kernel,
        out_type=jax.ShapeDtypeStruct((B,), jnp.float32),
        mesh=mesh,
        compiler_params=_sc_compiler_params(),
        scratch_types=[
            pltpu.VMEM((ROWS_PER_W,), jnp.int32),
            pltpu.VMEM((ROWS_PER_W,), jnp.int32),
            pltpu.VMEM((ROWS_PER_W,), jnp.int32),
            pltpu.VMEM((ROWS_PER_W,), jnp.int32),
            pltpu.VMEM((2, 16, 8, 64), jnp.float32),
            pltpu.VMEM((2, 16, 8, 64), jnp.float32),
            pltpu.VMEM((2, 16, 128), jnp.float32),
            pltpu.VMEM((16, PSTRIDE), jnp.float32),
            pltpu.VMEM((ROWS_PER_W,), jnp.float32),
            pltpu.SemaphoreType.DMA,
            pltpu.SemaphoreType.DMA,
        ],
    )
    def k(nodes_hbm, rel_hbm, hidx_hbm, tidx_hbm, rp_hbm, ro_hbm, out_hbm,
          h_v, t_v, rp_v, ro_v, hw, tw, rw, part, out_v, sem0, sem1):
        wid = lax.axis_index("s") * 2 + lax.axis_index("c")
        pltpu.sync_copy(hidx_hbm.at[wid], h_v)
        pltpu.sync_copy(tidx_hbm.at[wid], t_v)
        pltpu.sync_copy(rp_hbm.at[wid], rp_v)
        pltpu.sync_copy(ro_hbm.at[wid], ro_v)

        lanes = lax.iota(jnp.int32, 16)
        sems = (sem0, sem1)

        def issue_group(g, buf):
            sem = sems[buf]
            sl16 = pl.ds(g * 16, 16)
            hs = lax.shift_left(lax.shift_right_logical(h_v[sl16], 3), 3)
            ts = lax.shift_left(lax.shift_right_logical(t_v[sl16], 3), 3)
            for k2 in range(16):
                pltpu.async_copy(
                    nodes_hbm.at[pl.ds(pl.multiple_of(hs[k2], 8), 8)],
                    hw.at[buf, k2], sem)
                pltpu.async_copy(
                    nodes_hbm.at[pl.ds(pl.multiple_of(ts[k2], 8), 8)],
                    tw.at[buf, k2], sem)
            pltpu.async_copy(rel_hbm.at[rp_v.at[pl.ds(g * 16, 16)]],
                             rw.at[buf], sem)


        def drain_group(buf):
            sem = sems[buf]
            for k2 in range(16):
                pltpu.make_async_copy(
                    nodes_hbm.at[pl.ds(0, 8)], hw.at[buf, k2], sem).wait()
                pltpu.make_async_copy(
                    nodes_hbm.at[pl.ds(0, 8)], tw.at[buf, k2], sem).wait()
            pltpu.make_async_copy(
                rel_hbm.at[pl.ds(0, 16)], rw.at[buf], sem).wait()

        def compute_group(g, buf):
            sl16 = pl.ds(g * 16, 16)
            hsub = jnp.bitwise_and(h_v[sl16], 7)
            tsub = jnp.bitwise_and(t_v[sl16], 7)
            roff = ro_v[sl16]
            for k2 in range(16):
                sh = hsub[k2]
                st = tsub[k2]
                l0r = roff[k2]
                hh = tt = rr = hr = ht = rt = None
                for kk in range(4):
                    sl = pl.ds(kk * 16, 16)
                    h = hw[buf, k2, sh, sl]
                    t = tw[buf, k2, st, sl]
                    r = rw[buf, k2, pl.ds(l0r + kk * 16, 16)]
                    if kk == 0:
                        hh, tt, rr = h * h, t * t, r * r
                        hr, ht, rt = h * r, h * t, r * t
                    else:
                        hh += h * h
                        tt += t * t
                        rr += r * r
                        hr += h * r
                        ht += h * t
                        rt += r * t
                part[k2, pl.ds(0, 16)] = hh
                part[k2, pl.ds(16, 16)] = tt
                part[k2, pl.ds(32, 16)] = rr
                part[k2, pl.ds(48, 16)] = hr
                part[k2, pl.ds(64, 16)] = ht
                part[k2, pl.ds(80, 16)] = rt

            sums = []
            for q in range(6):
                acc = plsc.load_gather(
                    part, [lanes, jnp.full((16,), q * 16, jnp.int32)])
                for c in range(1, 16):
                    acc += plsc.load_gather(
                        part, [lanes, jnp.full((16,), q * 16 + c, jnp.int32)])
                sums.append(acc)
            shh, stt, srr, shr, sht, srt = sums

            a = jnp.minimum(_rsqrt16(shh), _BIG)
            b = jnp.minimum(_rsqrt16(stt), _BIG)
            dd = shh * a * a + srr + stt * b * b + 2.0 * (
                shr * a - sht * (a * b) - srt * b)
            dd = jnp.maximum(dd, 0.0)
            out_v[pl.ds(g * 16, 16)] = dd * jnp.minimum(_rsqrt16(dd), _BIG)

        issue_group(0, 0)

        @pl.loop(0, GROUPS // 2)
        def _(p):
            g0 = p * 2
            drain_group(0)
            issue_group(g0 + 1, 1)
            compute_group(g0, 0)
            drain_group(1)

            @pl.when(p < GROUPS // 2 - 1)
            def _():
                issue_group(g0 + 2, 0)

            compute_group(g0 + 1, 1)

        pltpu.sync_copy(out_v, out_hbm.at[pl.ds(wid * ROWS_PER_W, ROWS_PER_W)])

    return k(node_emb, rel2, hidx, tidx, rp, ro)
